# Initial kernel scaffold; baseline (speedup 1.0000x reference)
import jax
import jax.numpy as jnp
from jax.experimental import pallas as pl


def kernel(x, edge_index, Ws, bs):
    raise NotImplementedError("write your pallas kernel here")


# speedup vs baseline: 4.9461x; 4.9461x over previous
import functools

import jax
import jax.numpy as jnp
from jax import lax
from jax.experimental import pallas as pl
from jax.experimental.pallas import tpu as pltpu
from jax.experimental.pallas import tpu_sc as plsc

N = 10000
E = 320000
D = 128
NLAYERS = 8

NC, NS, L = 2, 16, 16
NW = NC * NS
NEXT = 10240
K = 128
NCH = 80
EPT = NCH * K
E_PAD = NW * EPT
RPT = NEXT // NS
BLK = 1024

_mesh = plsc.VectorSubcoreMesh(
    core_axis_name="c", subcore_axis_name="s", num_cores=NC, num_subcores=NS)


@functools.partial(
    pl.kernel,
    out_type=jax.ShapeDtypeStruct((NW, NEXT), jnp.float32),
    mesh=_mesh,
    scratch_types=[
        pltpu.VMEM((NCH, K), jnp.int32),
        pltpu.VMEM((NEXT,), jnp.float32),
    ],
    compiler_params=pltpu.CompilerParams(needs_layout_passes=False),
)
def _sc_degree(dstb_hbm, out_hbm, dst_v, loc):
    c = lax.axis_index("c")
    s = lax.axis_index("s")
    wid = c * NS + s
    zeros16 = jnp.zeros((L,), jnp.float32)

    def zbody(i, carry):
        loc[pl.ds(i * L, L)] = zeros16
        return carry
    lax.fori_loop(0, NEXT // L, zbody, 0)

    pltpu.sync_copy(dstb_hbm.at[wid], dst_v)
    ones16 = jnp.ones((L,), jnp.float32)

    def chunk(j, carry):
        for b in range(K // L):
            idx = dst_v[j, pl.ds(b * L, L)]
            plsc.addupdate_scatter(loc, [idx], ones16)
        return carry
    lax.fori_loop(0, NCH, chunk, 0)

    pltpu.sync_copy(loc, out_hbm.at[wid])


@functools.partial(
    pl.kernel,
    out_type=jax.ShapeDtypeStruct((NC, NEXT, D), jnp.float32),
    mesh=_mesh,
    scratch_types=[
        pltpu.VMEM((NCH, K), jnp.int32),
        pltpu.VMEM((NCH, K), jnp.int32),
        pltpu.VMEM((K, D), jnp.float32),
        pltpu.VMEM((L, D), jnp.float32),
        pltpu.VMEM_SHARED((NEXT, D), jnp.float32),
        pltpu.SemaphoreType.DMA,
    ],
)
def _sc_aggregate(g_hbm, srcb_hbm, dstb_hbm, out_hbm,
                  src_v, dst_v, rows_v, zrow_v, acc, gsem):
    c = lax.axis_index("c")
    s = lax.axis_index("s")
    wid = c * NS + s

    zeros16 = jnp.zeros((L,), jnp.float32)
    for i in range(L):
        for j in range(D // L):
            zrow_v[i, pl.ds(j * L, L)] = zeros16

    def zacc(i, carry):
        pltpu.sync_copy(zrow_v, acc.at[pl.ds(s * RPT + i * L, L)])
        return carry
    lax.fori_loop(0, RPT // L, zacc, 0)

    pltpu.sync_copy(srcb_hbm.at[wid], src_v)
    pltpu.sync_copy(dstb_hbm.at[wid], dst_v)
    plsc.subcore_barrier()

    def chunk(j, carry):
        pltpu.async_copy(g_hbm.at[src_v.at[j]], rows_v, gsem).wait()
        pltpu.sync_copy(rows_v, acc.at[dst_v.at[j]], add=True)
        return carry
    lax.fori_loop(0, NCH, chunk, 0)

    plsc.subcore_barrier()

    def wout(i, carry):
        r0 = s * RPT + i * 160
        pltpu.sync_copy(acc.at[pl.ds(r0, 160)], out_hbm.at[c, pl.ds(r0, 160)])
        return carry
    lax.fori_loop(0, RPT // 160, wout, 0)


def _dinv_body(degp_ref, o_ref):
    deg = jnp.sum(degp_ref[...], axis=0) + 1.0
    o_ref[...] = lax.rsqrt(deg)


_tc_dinv = pl.pallas_call(
    _dinv_body,
    out_shape=jax.ShapeDtypeStruct((NEXT,), jnp.float32),
)


def _prep_body(x_ref, dinv_ref, w_ref, o_ref):
    o_ref[...] = dinv_ref[...] * jnp.dot(
        x_ref[...], w_ref[...], preferred_element_type=jnp.float32)


_tc_prep = pl.pallas_call(
    _prep_body,
    grid=(NEXT // BLK,),
    in_specs=[
        pl.BlockSpec((BLK, D), lambda m: (m, 0)),
        pl.BlockSpec((BLK, 1), lambda m: (m, 0)),
        pl.BlockSpec((D, D), lambda m: (0, 0)),
    ],
    out_specs=pl.BlockSpec((BLK, D), lambda m: (m, 0)),
    out_shape=jax.ShapeDtypeStruct((NEXT, D), jnp.float32),
)


def _layer_body(p_ref, g_ref, dinv_ref, b_ref, w_ref, o_ref):
    t = p_ref[0] + p_ref[1] + g_ref[...]
    h = jnp.maximum(dinv_ref[...] * t + b_ref[...], 0.0)
    o_ref[...] = dinv_ref[...] * jnp.dot(
        h, w_ref[...], preferred_element_type=jnp.float32)


_tc_layer = pl.pallas_call(
    _layer_body,
    grid=(NEXT // BLK,),
    in_specs=[
        pl.BlockSpec((NC, BLK, D), lambda m: (0, m, 0)),
        pl.BlockSpec((BLK, D), lambda m: (m, 0)),
        pl.BlockSpec((BLK, 1), lambda m: (m, 0)),
        pl.BlockSpec((1, D), lambda m: (0, 0)),
        pl.BlockSpec((D, D), lambda m: (0, 0)),
    ],
    out_specs=pl.BlockSpec((BLK, D), lambda m: (m, 0)),
    out_shape=jax.ShapeDtypeStruct((NEXT, D), jnp.float32),
)


def _final_body(p_ref, g_ref, dinv_ref, b_ref, o_ref):
    t = p_ref[0] + p_ref[1] + g_ref[...]
    o_ref[...] = jax.nn.sigmoid(dinv_ref[...] * t + b_ref[...])


_tc_final = pl.pallas_call(
    _final_body,
    grid=(NEXT // BLK,),
    in_specs=[
        pl.BlockSpec((NC, BLK, D), lambda m: (0, m, 0)),
        pl.BlockSpec((BLK, D), lambda m: (m, 0)),
        pl.BlockSpec((BLK, 1), lambda m: (m, 0)),
        pl.BlockSpec((1, D), lambda m: (0, 0)),
    ],
    out_specs=pl.BlockSpec((BLK, D), lambda m: (m, 0)),
    out_shape=jax.ShapeDtypeStruct((NEXT, D), jnp.float32),
)


def kernel(x, edge_index, Ws, bs):
    src = edge_index[0].astype(jnp.int32)
    dst = edge_index[1].astype(jnp.int32)
    pad = jnp.full((E_PAD - E,), N, jnp.int32)
    srcb = jnp.concatenate([src, pad]).reshape(NW, NCH, K)
    dstb = jnp.concatenate([dst, pad]).reshape(NW, NCH, K)
    x_pad = jnp.zeros((NEXT, D), jnp.float32).at[:N].set(x)

    degp = _sc_degree(dstb)
    dinv = _tc_dinv(degp)[:, None]

    g = _tc_prep(x_pad, dinv, Ws[0])
    for i in range(NLAYERS):
        p = _sc_aggregate(g, srcb, dstb)
        if i < NLAYERS - 1:
            g = _tc_layer(p, g, dinv, bs[i][None, :], Ws[i + 1])
        else:
            out = _tc_final(p, g, dinv, bs[i][None, :])
    return out[:N]
